# baseline (speedup 1.0000x reference)
import functools

import numpy as np

import jax
import jax.numpy as jnp
from jax import lax
from jax.experimental import pallas as pl
from jax.experimental.pallas import tpu as pltpu
from jax.experimental.pallas import tpu_sc as plsc

N = 10000
E = 160000
D = 128
R = 200
RD = 32
NLAYERS = 3
NB = 2
B = 100

NC = 2
NS = 16
NW = NC * NS

CHUNK = 128
ECHUNK = 64
N_CHUNKS = 162
E_PER_T = N_CHUNKS * ECHUNK
EPAD = NS * E_PER_T
NSLOT = 3
DH = D // 2
WDEG = DH + 16

NPAD = 10112
ROWS_PER_TILE = NPAD // NS
RPAD = 208
BPAD = 128
IPAD = 104
NPAD2 = 12288
NODES_PER_W = NPAD2 // NW
POOL_CHUNKS = NODES_PER_W // CHUNK

F32 = jnp.float32
BF16 = jnp.bfloat16
I32 = jnp.int32

_PERM = np.array([g * 32 + p * 16 + i
                  for g in range(4) for i in range(16) for p in range(2)])

_MESH = plsc.VectorSubcoreMesh(core_axis_name="c", subcore_axis_name="s")
_SC_PARAMS = pltpu.CompilerParams(use_tc_tiling_on_sc=False,
                                  needs_layout_passes=False)


def _zero_rows(zsrc, dst_sh, base, total, zrows):
    off = 0
    while off < total:
        sz = min(zrows, total - off)
        pltpu.sync_copy(zsrc.at[pl.ds(0, sz)], dst_sh.at[pl.ds(base + off, sz)])
        off += sz


def _edge_body(compute_deg, *refs):
    w = WDEG if compute_deg else DH
    (hbT, src_h, dst_h, et_h, ct_h, aggp,
     agg_sh, srcv, dstv, etv, *rest) = refs

    cid = lax.axis_index("c")
    sid = lax.axis_index("s")

    gbufs = tuple(rest[0:NSLOT])
    cbufs = tuple(rest[NSLOT:2 * NSLOT])
    mbufs = tuple(rest[2 * NSLOT:3 * NSLOT])
    sgs = tuple(rest[3 * NSLOT:4 * NSLOT])
    sss = tuple(rest[4 * NSLOT:5 * NSLOT])
    mb0 = mbufs[0]

    z16 = jnp.zeros((16,), F32)
    o16 = jnp.ones((16,), F32)

    @plsc.parallel_loop(0, ECHUNK, unroll=2)
    def _(i):
        for j in range(w // 16):
            for mb in mbufs:
                mb[i, pl.ds(j * 16, 16)] = z16

    base = sid * ROWS_PER_TILE
    _zero_rows(mb0, agg_sh, base, ROWS_PER_TILE, ECHUNK)
    if compute_deg:
        @plsc.parallel_loop(0, ECHUNK, unroll=2)
        def _(i):
            for mb in mbufs:
                mb[i, pl.ds(DH, 16)] = o16
    plsc.subcore_barrier()

    r0 = pl.multiple_of(sid * N_CHUNKS, N_CHUNKS)
    pltpu.sync_copy(src_h.at[cid].at[pl.ds(r0, N_CHUNKS)], srcv)
    pltpu.sync_copy(dst_h.at[pl.ds(r0, N_CHUNKS)], dstv)
    pltpu.sync_copy(et_h.at[pl.ds(r0, N_CHUNKS)], etv)

    def issue_g(c, b):
        pltpu.async_copy(hbT.at[srcv.at[c]], gbufs[b], sgs[b])
        pltpu.async_copy(ct_h.at[etv.at[c]], cbufs[b], sgs[b])

    def wait_g(b):
        pltpu.make_async_copy(hbT.at[pl.ds(0, ECHUNK)], gbufs[b], sgs[b]).wait()
        pltpu.make_async_copy(ct_h.at[pl.ds(0, ECHUNK)], cbufs[b], sgs[b]).wait()

    def issue_s(c, b):
        pltpu.async_copy(mbufs[b], agg_sh.at[dstv.at[c]], sss[b], add=True)

    def wait_s(b):
        pltpu.make_async_copy(aggp.at[0].at[pl.ds(0, ECHUNK)],
                              mbufs[b], sss[b]).wait()

    def compute(b):
        gb = gbufs[b]
        cb = cbufs[b]
        mb = mbufs[b]

        @plsc.parallel_loop(0, ECHUNK, unroll=2)
        def _(i):
            ca, cc = plsc.unpack(cb[i, pl.ds(0, 32)],
                                 format=plsc.PackFormat.INTERLEAVED)
            for g in range(DH // 32):
                v0 = gb[i, pl.ds(g * 32, 32)]
                v1 = gb[i, pl.ds(DH + g * 32, 32)]
                a0, a1 = plsc.unpack(v0, format=plsc.PackFormat.INTERLEAVED)
                b0, b1 = plsc.unpack(v1, format=plsc.PackFormat.INTERLEAVED)
                mb[i, pl.ds(g * 32, 16)] = a0 * ca + b0 * cc
                mb[i, pl.ds(g * 32 + 16, 16)] = a1 * ca + b1 * cc

    for b in range(NSLOT):
        issue_g(b, b)

    def group_body(kk, _):
        for b in range(NSLOT):
            c = NSLOT * kk + b
            wait_g(b)

            @pl.when(kk > 0)
            def _():
                wait_s(b)

            compute(b)
            issue_g(jnp.minimum(c + NSLOT, N_CHUNKS - 1), b)
            issue_s(c, b)
        return _

    lax.fori_loop(0, N_CHUNKS // NSLOT, group_body, None)
    for b in range(NSLOT):
        wait_g(b)
        wait_s(b)
    plsc.subcore_barrier()

    rows = pl.ds(base, ROWS_PER_TILE)
    pltpu.sync_copy(agg_sh.at[rows], aggp.at[cid].at[rows])


def _make_edge_kernel(compute_deg):
    w = WDEG if compute_deg else DH
    return pl.kernel(
        functools.partial(_edge_body, compute_deg),
        out_type=(jax.ShapeDtypeStruct((NC, NPAD, w), F32),),
        mesh=_MESH,
        compiler_params=_SC_PARAMS,
        scratch_types=[
            pltpu.VMEM_SHARED((NPAD, w), F32),
            pltpu.VMEM((N_CHUNKS, ECHUNK), I32),
            pltpu.VMEM((N_CHUNKS, ECHUNK), I32),
            pltpu.VMEM((N_CHUNKS, ECHUNK), I32),
        ] + [pltpu.VMEM((ECHUNK, D), BF16)] * NSLOT
          + [pltpu.VMEM((ECHUNK, 32), BF16)] * NSLOT
          + [pltpu.VMEM((ECHUNK, w), F32)] * NSLOT
          + [pltpu.SemaphoreType.DMA] * (2 * NSLOT),
    )


ROWS_BLK = 1000
GRID_N = N // ROWS_BLK


def _write_tables(hbT_ref, hw_ref, h, m_ref):
    y0 = jnp.dot(h, m_ref[0], preferred_element_type=F32)
    y1 = jnp.dot(h, m_ref[1], preferred_element_type=F32)
    hbT_ref[0] = jnp.concatenate([y0[:, :DH], y1[:, :DH]], axis=1).astype(BF16)
    hbT_ref[1] = jnp.concatenate([y0[:, DH:], y1[:, DH:]], axis=1).astype(BF16)
    hw_ref[...] = jnp.dot(h, m_ref[2], preferred_element_type=F32)


def _transform_body(h_ref, m_ref, hbT_ref, hw_ref):
    _write_tables(hbT_ref, hw_ref, h_ref[...], m_ref)


def _transform(h, m):
    return pl.pallas_call(
        _transform_body,
        grid=(GRID_N,),
        in_specs=[
            pl.BlockSpec((ROWS_BLK, D), lambda i: (i, 0)),
            pl.BlockSpec((3, D, D), lambda i: (0, 0, 0)),
        ],
        out_specs=[pl.BlockSpec((2, ROWS_BLK, D), lambda i: (0, i, 0)),
                   pl.BlockSpec((ROWS_BLK, D), lambda i: (i, 0))],
        out_shape=[jax.ShapeDtypeStruct((2, N, D), BF16),
                   jax.ShapeDtypeStruct((N, D), F32)],
    )(h, m)


def _combine_body(has_next, agg_ref, deg_ref, hw_ref, b_ref, m_ref,
                  h_ref, *next_refs):
    agg = jnp.concatenate([agg_ref[0, :, 0:DH], agg_ref[1, :, 0:DH]], axis=1)
    deg = jnp.maximum(deg_ref[0, :, DH:DH + 1], 1.0)
    h = jnp.maximum(agg / deg + hw_ref[...] + b_ref[...], 0.0)
    h_ref[...] = h
    if has_next:
        nhbT, nhw = next_refs
        _write_tables(nhbT, nhw, h, m_ref)


def _combine(aggp, degsrc, hw, bias_l, m_next):
    has_next = m_next is not None
    w = aggp.shape[-1]
    if m_next is None:
        m_next = jnp.zeros((3, D, D), F32)
    out_specs = [pl.BlockSpec((ROWS_BLK, D), lambda i: (i, 0))]
    out_shape = [jax.ShapeDtypeStruct((N, D), F32)]
    if has_next:
        out_specs += [pl.BlockSpec((2, ROWS_BLK, D), lambda i: (0, i, 0)),
                      pl.BlockSpec((ROWS_BLK, D), lambda i: (i, 0))]
        out_shape += [jax.ShapeDtypeStruct((2, N, D), BF16),
                      jax.ShapeDtypeStruct((N, D), F32)]
    return pl.pallas_call(
        functools.partial(_combine_body, has_next),
        grid=(GRID_N,),
        in_specs=[
            pl.BlockSpec((NC, ROWS_BLK, w), lambda i: (0, i, 0)),
            pl.BlockSpec((1, ROWS_BLK, WDEG), lambda i: (0, i, 0)),
            pl.BlockSpec((ROWS_BLK, D), lambda i: (i, 0)),
            pl.BlockSpec((1, D), lambda i: (0, 0)),
            pl.BlockSpec((3, D, D), lambda i: (0, 0, 0)),
        ],
        out_specs=out_specs,
        out_shape=out_shape,
    )(aggp, degsrc, hw, bias_l, m_next)


def _pool_body(h1, h2, h3, gid_h, head_h, tail_h, rel_h, relt_h,
               gs1p, gs2p, gs3p, cntp, hd1, hd2, hd3, tl1, tl2, tl3, rele,
               gs1, gs2, gs3, cnt_sh, buf, gidv, idxv, brel, ones, sem0):
    cid = lax.axis_index("c")
    sid = lax.axis_index("s")
    wid = sid * NC + cid

    z16 = jnp.zeros((16,), F32)
    o16 = jnp.ones((16,), F32)

    def fill_zero(i, _):
        for j in range(D // 16):
            buf[i, pl.ds(j * 16, 16)] = z16
        ones[i, pl.ds(0, 16)] = z16
        return _

    lax.fori_loop(0, CHUNK, fill_zero, None)

    base = sid * (BPAD // NS)
    for sh in (gs1, gs2, gs3):
        pltpu.sync_copy(buf.at[pl.ds(0, BPAD // NS)], sh.at[pl.ds(base, BPAD // NS)])
    pltpu.sync_copy(ones.at[pl.ds(0, BPAD // NS)], cnt_sh.at[pl.ds(base, BPAD // NS)])

    def fill_one(i, _):
        ones[i, pl.ds(0, 16)] = o16
        return _

    lax.fori_loop(0, CHUNK, fill_one, None)
    plsc.subcore_barrier()

    nbase = wid * NODES_PER_W
    for c in range(POOL_CHUNKS):
        b0 = pl.multiple_of(nbase + c * CHUNK, CHUNK)
        pltpu.sync_copy(gid_h.at[pl.ds(b0, CHUNK)], gidv)
        for (h_t, g_sh) in ((h1, gs1), (h2, gs2), (h3, gs3)):
            pltpu.sync_copy(h_t.at[pl.ds(b0, CHUNK)], buf)
            pltpu.sync_copy(buf, g_sh.at[gidv], add=True)
        pltpu.sync_copy(ones, cnt_sh.at[gidv], add=True)

    tasks = ((head_h, h1, hd1), (head_h, h2, hd2), (head_h, h3, hd3),
             (tail_h, h1, tl1), (tail_h, h2, tl2), (tail_h, h3, tl3))
    for t, (ids_h, tab, out) in enumerate(tasks):
        @pl.when(wid == t)
        def _():
            pltpu.sync_copy(ids_h, idxv)
            pltpu.async_copy(tab.at[idxv], buf.at[pl.ds(0, IPAD)], sem0).wait()
            pltpu.sync_copy(buf.at[pl.ds(0, IPAD)], out)

    @pl.when(wid == 6)
    def _():
        pltpu.sync_copy(rel_h, idxv)
        pltpu.async_copy(relt_h.at[idxv], brel, sem0).wait()
        pltpu.sync_copy(brel, rele)

    plsc.subcore_barrier()
    rows = pl.ds(base, BPAD // NS)
    pltpu.sync_copy(gs1.at[rows], gs1p.at[cid].at[rows])
    pltpu.sync_copy(gs2.at[rows], gs2p.at[cid].at[rows])
    pltpu.sync_copy(gs3.at[rows], gs3p.at[cid].at[rows])
    pltpu.sync_copy(cnt_sh.at[rows], cntp.at[cid].at[rows])


_pool_kernel = pl.kernel(
    _pool_body,
    out_type=(
        jax.ShapeDtypeStruct((NC, BPAD, D), F32),
        jax.ShapeDtypeStruct((NC, BPAD, D), F32),
        jax.ShapeDtypeStruct((NC, BPAD, D), F32),
        jax.ShapeDtypeStruct((NC, BPAD, 16), F32),
        jax.ShapeDtypeStruct((IPAD, D), F32),
        jax.ShapeDtypeStruct((IPAD, D), F32),
        jax.ShapeDtypeStruct((IPAD, D), F32),
        jax.ShapeDtypeStruct((IPAD, D), F32),
        jax.ShapeDtypeStruct((IPAD, D), F32),
        jax.ShapeDtypeStruct((IPAD, D), F32),
        jax.ShapeDtypeStruct((IPAD, RD), F32),
    ),
    mesh=_MESH,
    compiler_params=_SC_PARAMS,
    scratch_types=[
        pltpu.VMEM_SHARED((BPAD, D), F32),
        pltpu.VMEM_SHARED((BPAD, D), F32),
        pltpu.VMEM_SHARED((BPAD, D), F32),
        pltpu.VMEM_SHARED((BPAD, 16), F32),
        pltpu.VMEM((CHUNK, D), F32),
        pltpu.VMEM((CHUNK,), I32),
        pltpu.VMEM((IPAD,), I32),
        pltpu.VMEM((IPAD, RD), F32),
        pltpu.VMEM((CHUNK, 16), F32),
        pltpu.SemaphoreType.DMA,
    ],
)


def _final_body(gs1_ref, gs2_ref, gs3_ref, cnt_ref,
                hd1_ref, hd2_ref, hd3_ref, tl1_ref, tl2_ref, tl3_ref,
                rel_ref, w_ref, b_ref, out_ref):
    cnt = jnp.maximum(cnt_ref[0, :, 0:1] + cnt_ref[1, :, 0:1], 1.0)
    acc = jnp.zeros((BPAD, 1), F32)
    for i, gref in enumerate((gs1_ref, gs2_ref, gs3_ref)):
        g = (gref[0] + gref[1]) / cnt
        acc = acc + jnp.dot(g, w_ref[pl.ds(i * D, D)],
                            preferred_element_type=F32)
    acc = acc[0:IPAD]
    for i, href in enumerate((hd1_ref, hd2_ref, hd3_ref)):
        acc = acc + jnp.dot(href[...], w_ref[pl.ds(384 + i * D, D)],
                            preferred_element_type=F32)
    for i, tref in enumerate((tl1_ref, tl2_ref, tl3_ref)):
        acc = acc + jnp.dot(tref[...], w_ref[pl.ds(768 + i * D, D)],
                            preferred_element_type=F32)
    acc = acc + jnp.dot(rel_ref[...], w_ref[pl.ds(1152, RD)],
                        preferred_element_type=F32)
    out_ref[...] = jnp.broadcast_to(acc + b_ref[0, 0], (IPAD, D))


def _final(gs1p, gs2p, gs3p, cntp, hd1, hd2, hd3, tl1, tl2, tl3, rele, fcW, fcb):
    return pl.pallas_call(
        _final_body,
        out_shape=jax.ShapeDtypeStruct((IPAD, D), F32),
    )(gs1p, gs2p, gs3p, cntp, hd1, hd2, hd3, tl1, tl2, tl3, rele, fcW, fcb)


def kernel(x, edge_index, edge_type, graph_ids, head_ids, tail_ids, rel_labels,
           basis, comp, Wself, bias, rel_table, fcW, fcb):
    src1 = jnp.concatenate([edge_index[0], jnp.zeros((EPAD - E,), I32)])
    src = jnp.stack([src1, src1 + N]).reshape(NC, EPAD // ECHUNK, ECHUNK)
    dst = jnp.concatenate([edge_index[1], jnp.full((EPAD - E,), N, I32)]
                          ).reshape(EPAD // ECHUNK, ECHUNK)
    et = jnp.concatenate([edge_type, jnp.zeros((EPAD - E,), I32)]
                         ).reshape(EPAD // ECHUNK, ECHUNK)

    perm = jnp.asarray(_PERM)
    ms = [jnp.concatenate([basis[l][:, :, perm], Wself[l][None]], axis=0)
          for l in range(NLAYERS)]
    cts = [jnp.pad(
        jnp.tile(comp[l][:, None, :], (1, 16, 1)).reshape(R, 32),
        ((0, RPAD - R), (0, 0))).astype(BF16) for l in range(NLAYERS)]

    edge_deg = _make_edge_kernel(True)
    edge_plain = _make_edge_kernel(False)

    hbT2, hw = _transform(x, ms[0])
    (agg1,) = edge_deg(hbT2.reshape(2 * N, D), src, dst, et, cts[0])
    h1, hbT2, hw = _combine(agg1, agg1, hw, bias[0][None], ms[1])
    (aggp,) = edge_plain(hbT2.reshape(2 * N, D), src, dst, et, cts[1])
    h2, hbT2, hw = _combine(aggp, agg1, hw, bias[1][None], ms[2])
    (aggp,) = edge_plain(hbT2.reshape(2 * N, D), src, dst, et, cts[2])
    (h3,) = _combine(aggp, agg1, hw, bias[2][None], None)

    pad_n = NPAD2 - N
    h1p = jnp.pad(h1, ((0, pad_n), (0, 0)))
    h2p = jnp.pad(h2, ((0, pad_n), (0, 0)))
    h3p = jnp.pad(h3, ((0, pad_n), (0, 0)))
    gidp = jnp.concatenate([graph_ids, jnp.full((pad_n,), BPAD - 1, I32)])
    headp = jnp.pad(head_ids, (0, IPAD - B))
    tailp = jnp.pad(tail_ids, (0, IPAD - B))
    relp = jnp.pad(rel_labels, (0, IPAD - B))

    outs = _pool_kernel(h1p, h2p, h3p, gidp, headp, tailp, relp, rel_table)
    final = _final(*outs, fcW, fcb[None])
    return final[:B, 0:1]


# speedup vs baseline: 1.1741x; 1.1741x over previous
import functools

import numpy as np

import jax
import jax.numpy as jnp
from jax import lax
from jax.experimental import pallas as pl
from jax.experimental.pallas import tpu as pltpu
from jax.experimental.pallas import tpu_sc as plsc

N = 10000
E = 160000
D = 128
R = 200
RD = 32
NLAYERS = 3
NB = 2
B = 100

NC = 2
NS = 16
NW = NC * NS

CHUNK = 128
ECHUNK = 64
N_CHUNKS = 160
E_PER_T = N_CHUNKS * ECHUNK
EPAD = NS * E_PER_T
NSLOT = 2
DH = D // 2
WDEG = DH + 16

NPAD = 10112
ROWS_PER_TILE = NPAD // NS
RPAD = 208
BPAD = 128
IPAD = 104
NPAD2 = 12288
NODES_PER_W = NPAD2 // NW
POOL_CHUNKS = NODES_PER_W // CHUNK

F32 = jnp.float32
BF16 = jnp.bfloat16
I32 = jnp.int32

_PERM = np.array([g * 32 + p * 16 + i
                  for g in range(4) for i in range(16) for p in range(2)])

_MESH = plsc.VectorSubcoreMesh(core_axis_name="c", subcore_axis_name="s")
_SC_PARAMS = pltpu.CompilerParams(use_tc_tiling_on_sc=False,
                                  needs_layout_passes=False)


def _zero_rows(zsrc, dst_sh, base, total, zrows):
    off = 0
    while off < total:
        sz = min(zrows, total - off)
        pltpu.sync_copy(zsrc.at[pl.ds(0, sz)], dst_sh.at[pl.ds(base + off, sz)])
        off += sz


def _edge_body(compute_deg, *refs):
    w = WDEG if compute_deg else DH
    (hbT, src_h, dst_h, et_h, ct_h, aggp,
     agg_sh, srcv, dstv, etv, *rest) = refs

    cid = lax.axis_index("c")
    sid = lax.axis_index("s")

    gbufs = tuple(rest[0:NSLOT])
    cbufs = tuple(rest[NSLOT:2 * NSLOT])
    mbufs = tuple(rest[2 * NSLOT:3 * NSLOT])
    sgs = tuple(rest[3 * NSLOT:4 * NSLOT])
    sss = tuple(rest[4 * NSLOT:5 * NSLOT])
    mb0 = mbufs[0]

    z16 = jnp.zeros((16,), F32)
    o16 = jnp.ones((16,), F32)

    @plsc.parallel_loop(0, ECHUNK, unroll=2)
    def _(i):
        for j in range(w // 16):
            for mb in mbufs:
                mb[i, pl.ds(j * 16, 16)] = z16

    base = sid * ROWS_PER_TILE
    _zero_rows(mb0, agg_sh, base, ROWS_PER_TILE, ECHUNK)
    if compute_deg:
        @plsc.parallel_loop(0, ECHUNK, unroll=2)
        def _(i):
            for mb in mbufs:
                mb[i, pl.ds(DH, 16)] = o16
    plsc.subcore_barrier()

    r0 = pl.multiple_of(sid * N_CHUNKS, N_CHUNKS)
    pltpu.sync_copy(src_h.at[cid].at[pl.ds(r0, N_CHUNKS)], srcv)
    pltpu.sync_copy(dst_h.at[pl.ds(r0, N_CHUNKS)], dstv)
    pltpu.sync_copy(et_h.at[pl.ds(r0, N_CHUNKS)], etv)

    def issue_g(c, b):
        pltpu.async_copy(hbT.at[srcv.at[c]], gbufs[b], sgs[b])
        pltpu.async_copy(ct_h.at[etv.at[c]], cbufs[b], sgs[b])

    def wait_g(b):
        pltpu.make_async_copy(hbT.at[pl.ds(0, ECHUNK)], gbufs[b], sgs[b]).wait()
        pltpu.make_async_copy(ct_h.at[pl.ds(0, ECHUNK)], cbufs[b], sgs[b]).wait()

    def issue_s(c, b):
        pltpu.async_copy(mbufs[b], agg_sh.at[dstv.at[c]], sss[b], add=True)

    def wait_s(b):
        pltpu.make_async_copy(aggp.at[0].at[pl.ds(0, ECHUNK)],
                              mbufs[b], sss[b]).wait()

    def compute(b):
        gb = gbufs[b]
        cb = cbufs[b]
        mb = mbufs[b]

        @plsc.parallel_loop(0, ECHUNK, unroll=2)
        def _(i):
            ca, cc = plsc.unpack(cb[i, pl.ds(0, 32)],
                                 format=plsc.PackFormat.INTERLEAVED)
            for g in range(DH // 32):
                v0 = gb[i, pl.ds(g * 32, 32)]
                v1 = gb[i, pl.ds(DH + g * 32, 32)]
                a0, a1 = plsc.unpack(v0, format=plsc.PackFormat.INTERLEAVED)
                b0, b1 = plsc.unpack(v1, format=plsc.PackFormat.INTERLEAVED)
                mb[i, pl.ds(g * 32, 16)] = a0 * ca + b0 * cc
                mb[i, pl.ds(g * 32 + 16, 16)] = a1 * ca + b1 * cc

    for b in range(NSLOT):
        issue_g(b, b)

    def group_body(kk, _):
        for b in range(NSLOT):
            c = NSLOT * kk + b
            wait_g(b)

            @pl.when(kk > 0)
            def _():
                wait_s(b)

            compute(b)
            issue_g(jnp.minimum(c + NSLOT, N_CHUNKS - 1), b)
            issue_s(c, b)
        return _

    lax.fori_loop(0, N_CHUNKS // NSLOT, group_body, None)
    for b in range(NSLOT):
        wait_g(b)
        wait_s(b)
    plsc.subcore_barrier()

    rows = pl.ds(base, ROWS_PER_TILE)
    pltpu.sync_copy(agg_sh.at[rows], aggp.at[cid].at[rows])


def _make_edge_kernel(compute_deg):
    w = WDEG if compute_deg else DH
    return pl.kernel(
        functools.partial(_edge_body, compute_deg),
        out_type=(jax.ShapeDtypeStruct((NC, NPAD, w), F32),),
        mesh=_MESH,
        compiler_params=_SC_PARAMS,
        scratch_types=[
            pltpu.VMEM_SHARED((NPAD, w), F32),
            pltpu.VMEM((N_CHUNKS, ECHUNK), I32),
            pltpu.VMEM((N_CHUNKS, ECHUNK), I32),
            pltpu.VMEM((N_CHUNKS, ECHUNK), I32),
        ] + [pltpu.VMEM((ECHUNK, D), BF16)] * NSLOT
          + [pltpu.VMEM((ECHUNK, 32), BF16)] * NSLOT
          + [pltpu.VMEM((ECHUNK, w), F32)] * NSLOT
          + [pltpu.SemaphoreType.DMA] * (2 * NSLOT),
    )


ROWS_BLK = 1000
GRID_N = N // ROWS_BLK


def _write_tables(hbT_ref, hw_ref, h, m_ref):
    y0 = jnp.dot(h, m_ref[0], preferred_element_type=F32)
    y1 = jnp.dot(h, m_ref[1], preferred_element_type=F32)
    hbT_ref[0] = jnp.concatenate([y0[:, :DH], y1[:, :DH]], axis=1).astype(BF16)
    hbT_ref[1] = jnp.concatenate([y0[:, DH:], y1[:, DH:]], axis=1).astype(BF16)
    hw_ref[...] = jnp.dot(h, m_ref[2], preferred_element_type=F32)


def _transform_body(h_ref, m_ref, hbT_ref, hw_ref):
    _write_tables(hbT_ref, hw_ref, h_ref[...], m_ref)


def _transform(h, m):
    return pl.pallas_call(
        _transform_body,
        grid=(GRID_N,),
        in_specs=[
            pl.BlockSpec((ROWS_BLK, D), lambda i: (i, 0)),
            pl.BlockSpec((3, D, D), lambda i: (0, 0, 0)),
        ],
        out_specs=[pl.BlockSpec((2, ROWS_BLK, D), lambda i: (0, i, 0)),
                   pl.BlockSpec((ROWS_BLK, D), lambda i: (i, 0))],
        out_shape=[jax.ShapeDtypeStruct((2, N, D), BF16),
                   jax.ShapeDtypeStruct((N, D), F32)],
    )(h, m)


def _combine_body(has_next, agg_ref, deg_ref, hw_ref, b_ref, m_ref,
                  h_ref, *next_refs):
    agg = jnp.concatenate([agg_ref[0, :, 0:DH], agg_ref[1, :, 0:DH]], axis=1)
    deg = jnp.maximum(deg_ref[0, :, DH:DH + 1], 1.0)
    h = jnp.maximum(agg / deg + hw_ref[...] + b_ref[...], 0.0)
    h_ref[...] = h
    if has_next:
        nhbT, nhw = next_refs
        _write_tables(nhbT, nhw, h, m_ref)


def _combine(aggp, degsrc, hw, bias_l, m_next):
    has_next = m_next is not None
    w = aggp.shape[-1]
    if m_next is None:
        m_next = jnp.zeros((3, D, D), F32)
    out_specs = [pl.BlockSpec((ROWS_BLK, D), lambda i: (i, 0))]
    out_shape = [jax.ShapeDtypeStruct((N, D), F32)]
    if has_next:
        out_specs += [pl.BlockSpec((2, ROWS_BLK, D), lambda i: (0, i, 0)),
                      pl.BlockSpec((ROWS_BLK, D), lambda i: (i, 0))]
        out_shape += [jax.ShapeDtypeStruct((2, N, D), BF16),
                      jax.ShapeDtypeStruct((N, D), F32)]
    return pl.pallas_call(
        functools.partial(_combine_body, has_next),
        grid=(GRID_N,),
        in_specs=[
            pl.BlockSpec((NC, ROWS_BLK, w), lambda i: (0, i, 0)),
            pl.BlockSpec((1, ROWS_BLK, WDEG), lambda i: (0, i, 0)),
            pl.BlockSpec((ROWS_BLK, D), lambda i: (i, 0)),
            pl.BlockSpec((1, D), lambda i: (0, 0)),
            pl.BlockSpec((3, D, D), lambda i: (0, 0, 0)),
        ],
        out_specs=out_specs,
        out_shape=out_shape,
    )(aggp, degsrc, hw, bias_l, m_next)


def _pool_body(h1, h2, h3, gid_h, head_h, tail_h, rel_h, relt_h,
               gs1p, gs2p, gs3p, cntp, hd1, hd2, hd3, tl1, tl2, tl3, rele,
               gs1, gs2, gs3, cnt_sh, buf, gidv, idxv, brel, ones, sem0):
    cid = lax.axis_index("c")
    sid = lax.axis_index("s")
    wid = sid * NC + cid

    z16 = jnp.zeros((16,), F32)
    o16 = jnp.ones((16,), F32)

    def fill_zero(i, _):
        for j in range(D // 16):
            buf[i, pl.ds(j * 16, 16)] = z16
        ones[i, pl.ds(0, 16)] = z16
        return _

    lax.fori_loop(0, CHUNK, fill_zero, None)

    base = sid * (BPAD // NS)
    for sh in (gs1, gs2, gs3):
        pltpu.sync_copy(buf.at[pl.ds(0, BPAD // NS)], sh.at[pl.ds(base, BPAD // NS)])
    pltpu.sync_copy(ones.at[pl.ds(0, BPAD // NS)], cnt_sh.at[pl.ds(base, BPAD // NS)])

    def fill_one(i, _):
        ones[i, pl.ds(0, 16)] = o16
        return _

    lax.fori_loop(0, CHUNK, fill_one, None)
    plsc.subcore_barrier()

    nbase = wid * NODES_PER_W
    for c in range(POOL_CHUNKS):
        b0 = pl.multiple_of(nbase + c * CHUNK, CHUNK)
        pltpu.sync_copy(gid_h.at[pl.ds(b0, CHUNK)], gidv)
        for (h_t, g_sh) in ((h1, gs1), (h2, gs2), (h3, gs3)):
            pltpu.sync_copy(h_t.at[pl.ds(b0, CHUNK)], buf)
            pltpu.sync_copy(buf, g_sh.at[gidv], add=True)
        pltpu.sync_copy(ones, cnt_sh.at[gidv], add=True)

    tasks = ((head_h, h1, hd1), (head_h, h2, hd2), (head_h, h3, hd3),
             (tail_h, h1, tl1), (tail_h, h2, tl2), (tail_h, h3, tl3))
    for t, (ids_h, tab, out) in enumerate(tasks):
        @pl.when(wid == t)
        def _():
            pltpu.sync_copy(ids_h, idxv)
            pltpu.async_copy(tab.at[idxv], buf.at[pl.ds(0, IPAD)], sem0).wait()
            pltpu.sync_copy(buf.at[pl.ds(0, IPAD)], out)

    @pl.when(wid == 6)
    def _():
        pltpu.sync_copy(rel_h, idxv)
        pltpu.async_copy(relt_h.at[idxv], brel, sem0).wait()
        pltpu.sync_copy(brel, rele)

    plsc.subcore_barrier()
    rows = pl.ds(base, BPAD // NS)
    pltpu.sync_copy(gs1.at[rows], gs1p.at[cid].at[rows])
    pltpu.sync_copy(gs2.at[rows], gs2p.at[cid].at[rows])
    pltpu.sync_copy(gs3.at[rows], gs3p.at[cid].at[rows])
    pltpu.sync_copy(cnt_sh.at[rows], cntp.at[cid].at[rows])


_pool_kernel = pl.kernel(
    _pool_body,
    out_type=(
        jax.ShapeDtypeStruct((NC, BPAD, D), F32),
        jax.ShapeDtypeStruct((NC, BPAD, D), F32),
        jax.ShapeDtypeStruct((NC, BPAD, D), F32),
        jax.ShapeDtypeStruct((NC, BPAD, 16), F32),
        jax.ShapeDtypeStruct((IPAD, D), F32),
        jax.ShapeDtypeStruct((IPAD, D), F32),
        jax.ShapeDtypeStruct((IPAD, D), F32),
        jax.ShapeDtypeStruct((IPAD, D), F32),
        jax.ShapeDtypeStruct((IPAD, D), F32),
        jax.ShapeDtypeStruct((IPAD, D), F32),
        jax.ShapeDtypeStruct((IPAD, RD), F32),
    ),
    mesh=_MESH,
    compiler_params=_SC_PARAMS,
    scratch_types=[
        pltpu.VMEM_SHARED((BPAD, D), F32),
        pltpu.VMEM_SHARED((BPAD, D), F32),
        pltpu.VMEM_SHARED((BPAD, D), F32),
        pltpu.VMEM_SHARED((BPAD, 16), F32),
        pltpu.VMEM((CHUNK, D), F32),
        pltpu.VMEM((CHUNK,), I32),
        pltpu.VMEM((IPAD,), I32),
        pltpu.VMEM((IPAD, RD), F32),
        pltpu.VMEM((CHUNK, 16), F32),
        pltpu.SemaphoreType.DMA,
    ],
)


def _final_body(gs1_ref, gs2_ref, gs3_ref, cnt_ref,
                hd1_ref, hd2_ref, hd3_ref, tl1_ref, tl2_ref, tl3_ref,
                rel_ref, w_ref, b_ref, out_ref):
    cnt = jnp.maximum(cnt_ref[0, :, 0:1] + cnt_ref[1, :, 0:1], 1.0)
    acc = jnp.zeros((BPAD, 1), F32)
    for i, gref in enumerate((gs1_ref, gs2_ref, gs3_ref)):
        g = (gref[0] + gref[1]) / cnt
        acc = acc + jnp.dot(g, w_ref[pl.ds(i * D, D)],
                            preferred_element_type=F32)
    acc = acc[0:IPAD]
    for i, href in enumerate((hd1_ref, hd2_ref, hd3_ref)):
        acc = acc + jnp.dot(href[...], w_ref[pl.ds(384 + i * D, D)],
                            preferred_element_type=F32)
    for i, tref in enumerate((tl1_ref, tl2_ref, tl3_ref)):
        acc = acc + jnp.dot(tref[...], w_ref[pl.ds(768 + i * D, D)],
                            preferred_element_type=F32)
    acc = acc + jnp.dot(rel_ref[...], w_ref[pl.ds(1152, RD)],
                        preferred_element_type=F32)
    out_ref[...] = jnp.broadcast_to(acc + b_ref[0, 0], (IPAD, D))


def _final(gs1p, gs2p, gs3p, cntp, hd1, hd2, hd3, tl1, tl2, tl3, rele, fcW, fcb):
    return pl.pallas_call(
        _final_body,
        out_shape=jax.ShapeDtypeStruct((IPAD, D), F32),
    )(gs1p, gs2p, gs3p, cntp, hd1, hd2, hd3, tl1, tl2, tl3, rele, fcW, fcb)


def kernel(x, edge_index, edge_type, graph_ids, head_ids, tail_ids, rel_labels,
           basis, comp, Wself, bias, rel_table, fcW, fcb):
    src1 = jnp.concatenate([edge_index[0], jnp.zeros((EPAD - E,), I32)])
    src = jnp.stack([src1, src1 + N]).reshape(NC, EPAD // ECHUNK, ECHUNK)
    dst = jnp.concatenate([edge_index[1], jnp.full((EPAD - E,), N, I32)]
                          ).reshape(EPAD // ECHUNK, ECHUNK)
    et = jnp.concatenate([edge_type, jnp.zeros((EPAD - E,), I32)]
                         ).reshape(EPAD // ECHUNK, ECHUNK)

    perm = jnp.asarray(_PERM)
    ms = [jnp.concatenate([basis[l][:, :, perm], Wself[l][None]], axis=0)
          for l in range(NLAYERS)]
    cts = [jnp.pad(
        jnp.tile(comp[l][:, None, :], (1, 16, 1)).reshape(R, 32),
        ((0, RPAD - R), (0, 0))).astype(BF16) for l in range(NLAYERS)]

    edge_deg = _make_edge_kernel(True)
    edge_plain = _make_edge_kernel(False)

    hbT2, hw = _transform(x, ms[0])
    (agg1,) = edge_deg(hbT2.reshape(2 * N, D), src, dst, et, cts[0])
    h1, hbT2, hw = _combine(agg1, agg1, hw, bias[0][None], ms[1])
    (aggp,) = edge_plain(hbT2.reshape(2 * N, D), src, dst, et, cts[1])
    h2, hbT2, hw = _combine(aggp, agg1, hw, bias[1][None], ms[2])
    (aggp,) = edge_plain(hbT2.reshape(2 * N, D), src, dst, et, cts[2])
    (h3,) = _combine(aggp, agg1, hw, bias[2][None], None)

    pad_n = NPAD2 - N
    h1p = jnp.pad(h1, ((0, pad_n), (0, 0)))
    h2p = jnp.pad(h2, ((0, pad_n), (0, 0)))
    h3p = jnp.pad(h3, ((0, pad_n), (0, 0)))
    gidp = jnp.concatenate([graph_ids, jnp.full((pad_n,), BPAD - 1, I32)])
    headp = jnp.pad(head_ids, (0, IPAD - B))
    tailp = jnp.pad(tail_ids, (0, IPAD - B))
    relp = jnp.pad(rel_labels, (0, IPAD - B))

    outs = _pool_kernel(h1p, h2p, h3p, gidp, headp, tailp, relp, rel_table)
    final = _final(*outs, fcW, fcb[None])
    return final[:B, 0:1]
